# Initial kernel scaffold; baseline (speedup 1.0000x reference)
import jax
import jax.numpy as jnp
from jax.experimental import pallas as pl


def kernel(x, W1, b1, W2, b2, W3, b3, leaf_best):
    raise NotImplementedError("write your pallas kernel here")


# speedup vs baseline: 1.4013x; 1.4013x over previous
import jax
import jax.numpy as jnp
from jax.experimental import pallas as pl
from jax.experimental.pallas import tpu as pltpu


def _mlp_node_kernel(x_ref, w1_ref, b1_ref, w2_ref, b2_ref, w3_ref, b3_ref,
                     d_ref):
    x = x_ref[...]
    h = jnp.tanh(jnp.dot(x, w1_ref[0], preferred_element_type=jnp.float32)
                 + b1_ref[0])
    h = jnp.tanh(jnp.dot(h, w2_ref[0], preferred_element_type=jnp.float32)
                 + b2_ref[0])
    logits = (jnp.dot(h, w3_ref[0], preferred_element_type=jnp.float32)
              + b3_ref[0])
    cmp = (logits[:, 0:1] >= logits[:, 1:2]).astype(jnp.float32)
    d_ref[...] = cmp[None, None, :, :]


def _combine_kernel(d_ref, lb_ref, out_ref):
    d0 = d_ref[0, :, :, 0]
    d1 = d_ref[1, :, :, 0]
    d2 = d_ref[2, :, :, 0]
    i1 = 2.0 - d0
    da1 = jnp.where(d0 > 0.5, d1, d2)
    i2 = 2.0 * i1 + 2.0 - da1
    da2 = jnp.zeros_like(d0)
    for k in range(3, 7):
        da2 = jnp.where(i2 == float(k), d_ref[k, :, :, 0], da2)
    leaf = 2.0 * i2 + 2.0 - da2 - 7.0
    out = jnp.zeros_like(d0)
    for j in range(8):
        out = jnp.where(leaf == float(j), lb_ref[j], out)
    out_ref[...] = out


def kernel(x, W1, b1, W2, b2, W3, b3, leaf_best):
    N, F = x.shape
    NODES, _, H = W1.shape
    BN = 512
    NT = N // BN

    d = pl.pallas_call(
        _mlp_node_kernel,
        grid=(NODES, NT),
        in_specs=[
            pl.BlockSpec((BN, F), lambda n, t: (t, 0)),
            pl.BlockSpec((1, F, H), lambda n, t: (n, 0, 0)),
            pl.BlockSpec((1, 1, H), lambda n, t: (n, 0, 0)),
            pl.BlockSpec((1, H, H), lambda n, t: (n, 0, 0)),
            pl.BlockSpec((1, 1, H), lambda n, t: (n, 0, 0)),
            pl.BlockSpec((1, H, 2), lambda n, t: (n, 0, 0)),
            pl.BlockSpec((1, 1, 2), lambda n, t: (n, 0, 0)),
        ],
        out_specs=pl.BlockSpec((1, 1, BN, 1), lambda n, t: (n, t, 0, 0)),
        out_shape=jax.ShapeDtypeStruct((NODES, NT, BN, 1), jnp.float32),
        compiler_params=pltpu.CompilerParams(
            dimension_semantics=("arbitrary", "arbitrary")),
    )(x, W1, b1[:, None, :], W2, b2[:, None, :], W3, b3[:, None, :])

    out = pl.pallas_call(
        _combine_kernel,
        in_specs=[
            pl.BlockSpec(memory_space=pltpu.VMEM),
            pl.BlockSpec(memory_space=pltpu.SMEM),
        ],
        out_shape=jax.ShapeDtypeStruct((NT, BN), jnp.float32),
    )(d, leaf_best)
    return out.reshape(N)
